# baseline (speedup 1.0000x reference)
import functools

import jax
import jax.numpy as jnp
from jax import lax
from jax.experimental import pallas as pl
from jax.experimental.pallas import tpu as pltpu
from jax.experimental.pallas import tpu_sc as plsc

K = 300
KPAD = 320
NSLOT = 384
BN = 2000
NROW = 160
NPAD = NROW * 128


def _score_kernel(mem_ref, lg_ref, an_ref, w1_ref, b1_ref, w2_ref, b2_ref,
                  wp_ref, bp_ref, out_ref):
    m = mem_ref[0]
    h = jnp.maximum(
        jnp.dot(m, w1_ref[...], preferred_element_type=jnp.float32)
        + b1_ref[...], 0.0)
    proj = jnp.dot(h, w2_ref[...], preferred_element_type=jnp.float32) \
        + b2_ref[...]
    wh = jax.nn.sigmoid(an_ref[0][:, 2:4])
    area = wh[:, 0:1] * wh[:, 1:2]
    maskf = (area < 0.1).astype(jnp.float32)
    enh = m + proj * maskf
    sw = jax.nn.sigmoid(
        jnp.dot(enh, wp_ref[...], preferred_element_type=jnp.float32)
        + bp_ref[...])
    cls = jnp.max(lg_ref[0], axis=-1, keepdims=True)
    out_ref[0] = cls * (1.0 + sw * maskf)


def _select_kernel(s_ref, out_ref, posx_ref, hi_ref, lo_ref, acc_ref):
    S = jnp.maximum(s_ref[0], -3.0e38)
    bits = lax.bitcast_convert_type(S, jnp.int32)
    key = bits ^ (lax.shift_right_arithmetic(bits, 31) & jnp.int32(0x7FFFFFFF))

    base = jnp.where(jnp.sum((key >= 0).astype(jnp.int32)) >= K,
                     jnp.int32(0), jnp.int32(-2147483648))
    for b in range(30, -1, -1):
        cand = base + jnp.int32(1 << b)
        cnt = jnp.sum((key >= cand).astype(jnp.int32))
        base = jnp.where(cnt >= K, cand, base)
    T = base
    gt = key > T
    need_eq = (K - jnp.sum(gt.astype(jnp.int32))).astype(jnp.float32)

    io0 = lax.broadcasted_iota(jnp.int32, (128, 128), 0)
    io1 = lax.broadcasted_iota(jnp.int32, (128, 128), 1)
    U = (io0 < io1).astype(jnp.float32)
    I128 = (io0 == io1).astype(jnp.float32)
    r0 = lax.broadcasted_iota(jnp.int32, (NROW, NROW), 0)
    r1 = lax.broadcasted_iota(jnp.int32, (NROW, NROW), 1)
    Lst = (r1 < r0).astype(jnp.float32)

    hp = jax.lax.Precision.HIGHEST

    def ex_prefix(maskf):
        inrow = jnp.dot(maskf, U, preferred_element_type=jnp.float32,
                        precision=hp)
        rowoff = jnp.dot(Lst, jnp.sum(maskf, axis=1, keepdims=True),
                         preferred_element_type=jnp.float32, precision=hp)
        return inrow + rowoff

    eq = key == T
    eq_rank = ex_prefix(eq.astype(jnp.float32))
    sel = gt | (eq & (eq_rank < need_eq))
    self_f = sel.astype(jnp.float32)
    pos = ex_prefix(self_f)
    posx_ref[...] = jnp.where(sel, pos, 3.0e6)

    acc_ref[...] = jnp.zeros((3, NSLOT), jnp.float32)
    iota_slot = lax.broadcasted_iota(
        jnp.int32, (1, NSLOT), 1).astype(jnp.float32)
    lane_io = lax.broadcasted_iota(
        jnp.int32, (1, 128), 1).astype(jnp.float32)
    hi_ref[...] = lax.shift_right_arithmetic(key, 16).astype(jnp.float32)
    lo_ref[...] = (key & jnp.int32(0xFFFF)).astype(jnp.float32)

    def body(r, carry):
        prow = posx_ref[pl.ds(r, 1), :]
        pcol = jnp.sum(I128 * prow, axis=1, keepdims=True)
        oh = (pcol == iota_slot).astype(jnp.float32)
        lirow = lane_io + 128.0 * r.astype(jnp.float32)
        vals = jnp.concatenate(
            [hi_ref[pl.ds(r, 1), :], lo_ref[pl.ds(r, 1), :], lirow],
            axis=0)
        acc_ref[...] += jnp.dot(vals, oh,
                                preferred_element_type=jnp.float32,
                                precision=hp)
        return carry

    lax.fori_loop(0, NROW, body, 0)

    slot_io = lax.broadcasted_iota(jnp.int32, (1, NSLOT), 1)
    valid = slot_io < K
    hic = jnp.where(valid, acc_ref[0:1, :], -32768.0)
    loc = jnp.where(valid, acc_ref[1:2, :], 0.0)
    lic = jnp.where(valid, acc_ref[2:3, :], 4.0e6 + iota_slot)

    hcols, locols, lcols = [], [], []
    for c in range(NSLOT // 128):
        sl = slice(c * 128, (c + 1) * 128)
        hcols.append(jnp.sum(I128 * hic[:, sl], axis=1, keepdims=True))
        locols.append(jnp.sum(I128 * loc[:, sl], axis=1, keepdims=True))
        lcols.append(jnp.sum(I128 * lic[:, sl], axis=1, keepdims=True))
    hcol = jnp.concatenate(hcols, axis=0)
    locol = jnp.concatenate(locols, axis=0)
    lcol = jnp.concatenate(lcols, axis=0)

    keyr = (jnp.left_shift(hic.astype(jnp.int32), 16)
            | loc.astype(jnp.int32))
    keya = (jnp.left_shift(hcol.astype(jnp.int32), 16)
            | locol.astype(jnp.int32))

    before = (keyr > keya) | ((keyr == keya) & (lic < lcol))
    ranks = jnp.sum(before.astype(jnp.float32), axis=1, keepdims=True)
    oh2 = (ranks == iota_slot).astype(jnp.float32)
    ordered = jnp.sum(oh2 * lcol, axis=0, keepdims=True)

    pid = pl.program_id(0)
    out_ref[0] = (jnp.where(valid, ordered.astype(jnp.int32), 0)
                  + pid * 20000)


def _sc_gather(nb, idx_hbm, mem_hbm, la_hbm,
               om_hbm, ola_hbm, idx_v, mem_v, la_v, sem):
    wid = lax.axis_index("s") * 2 + lax.axis_index("c")
    base = wid * nb
    pltpu.sync_copy(idx_hbm.at[pl.ds(base, nb)], idx_v)
    pltpu.async_copy(mem_hbm.at[idx_v], mem_v, sem).wait()
    pltpu.async_copy(la_hbm.at[idx_v], la_v, sem).wait()
    pltpu.sync_copy(mem_v, om_hbm.at[pl.ds(base, nb)])
    pltpu.sync_copy(la_v, ola_hbm.at[pl.ds(base, nb)])


def kernel(memory, outputs_logits, anchors, W1, b1, W2, b2, Wp, bp, topk):
    bs, N, H = memory.shape
    C = outputs_logits.shape[-1]
    f32 = jnp.float32

    scores3 = pl.pallas_call(
        _score_kernel,
        grid=(bs, N // BN),
        in_specs=[
            pl.BlockSpec((1, BN, H), lambda b, i: (b, i, 0)),
            pl.BlockSpec((1, BN, C), lambda b, i: (b, i, 0)),
            pl.BlockSpec((1, BN, 4), lambda b, i: (b, i, 0)),
            pl.BlockSpec((H, H // 2), lambda b, i: (0, 0)),
            pl.BlockSpec((1, H // 2), lambda b, i: (0, 0)),
            pl.BlockSpec((H // 2, H), lambda b, i: (0, 0)),
            pl.BlockSpec((1, H), lambda b, i: (0, 0)),
            pl.BlockSpec((H, 1), lambda b, i: (0, 0)),
            pl.BlockSpec((1, 1), lambda b, i: (0, 0)),
        ],
        out_specs=pl.BlockSpec((1, BN, 1), lambda b, i: (b, i, 0)),
        out_shape=jax.ShapeDtypeStruct((bs, N, 1), f32),
    )(memory, outputs_logits, anchors,
      W1, b1.reshape(1, -1), W2, b2.reshape(1, -1), Wp, bp.reshape(1, 1))

    spad = jnp.concatenate(
        [scores3[..., 0], jnp.full((bs, NPAD - N), -jnp.inf, f32)],
        axis=1).reshape(bs, NROW, 128)

    idx = pl.pallas_call(
        _select_kernel,
        grid=(bs,),
        in_specs=[pl.BlockSpec((1, NROW, 128), lambda b: (b, 0, 0))],
        out_specs=pl.BlockSpec((1, 1, NSLOT), lambda b: (b, 0, 0)),
        out_shape=jax.ShapeDtypeStruct((bs, 1, NSLOT), jnp.int32),
        scratch_shapes=[pltpu.VMEM((NROW, 128), f32),
                        pltpu.VMEM((NROW, 128), f32),
                        pltpu.VMEM((NROW, 128), f32),
                        pltpu.VMEM((3, NSLOT), f32)],
    )(spad)

    idx_flat = idx[:, 0, :KPAD].reshape(bs * KPAD)

    la = jnp.pad(
        jnp.concatenate([outputs_logits.reshape(bs * N, C),
                         anchors.reshape(bs * N, 4)], axis=1),
        ((0, 0), (0, 128 - C - 4)))
    nw = 32
    nb = (bs * KPAD) // nw
    mesh = plsc.VectorSubcoreMesh(core_axis_name="c", subcore_axis_name="s")
    gath = functools.partial(
        pl.kernel, mesh=mesh,
        out_type=[jax.ShapeDtypeStruct((bs * KPAD, H), f32),
                  jax.ShapeDtypeStruct((bs * KPAD, 128), f32)],
        scratch_types=[pltpu.VMEM((nb,), jnp.int32),
                       pltpu.VMEM((nb, H), f32),
                       pltpu.VMEM((nb, 128), f32),
                       pltpu.SemaphoreType.DMA],
    )(functools.partial(_sc_gather, nb))
    om, ola = gath(idx_flat, memory.reshape(bs * N, H), la)

    topk_memory = om.reshape(bs, KPAD, H)[:, :K]
    topk_logits = ola.reshape(bs, KPAD, 128)[:, :K, :C]
    topk_anchors = ola.reshape(bs, KPAD, 128)[:, :K, C:C + 4]
    return (topk_memory, topk_logits, topk_anchors)


# speedup vs baseline: 1.4010x; 1.4010x over previous
import functools

import jax
import jax.numpy as jnp
from jax import lax
from jax.experimental import pallas as pl
from jax.experimental.pallas import tpu as pltpu
from jax.experimental.pallas import tpu_sc as plsc

K = 300
KPAD = 320
NSLOT = 384
BN = 2000
NROW = 160
NPAD = NROW * 128


def _score_kernel(mem_ref, lg_ref, an_ref, w1_ref, b1_ref, w2_ref, b2_ref,
                  wp_ref, bp_ref, out_ref):
    m = mem_ref[0]
    h = jnp.maximum(
        jnp.dot(m, w1_ref[...], preferred_element_type=jnp.float32)
        + b1_ref[...], 0.0)
    proj = jnp.dot(h, w2_ref[...], preferred_element_type=jnp.float32) \
        + b2_ref[...]
    wh = jax.nn.sigmoid(an_ref[0][:, 2:4])
    area = wh[:, 0:1] * wh[:, 1:2]
    maskf = (area < 0.1).astype(jnp.float32)
    enh = m + proj * maskf
    sw = jax.nn.sigmoid(
        jnp.dot(enh, wp_ref[...], preferred_element_type=jnp.float32)
        + bp_ref[...])
    cls = jnp.max(lg_ref[0], axis=-1, keepdims=True)
    out_ref[0] = cls * (1.0 + sw * maskf)


def _select_kernel(s_ref, out_ref, posx_ref, hi_ref, lo_ref, acc_ref):
    S = jnp.maximum(s_ref[0], -3.0e38)
    bits = lax.bitcast_convert_type(S, jnp.int32)
    key = bits ^ (lax.shift_right_arithmetic(bits, 31) & jnp.int32(0x7FFFFFFF))

    base = jnp.where(jnp.sum((key >= 0).astype(jnp.int32)) >= K,
                     jnp.int32(0), jnp.int32(-2147483648))
    for b in range(30, -1, -1):
        cand = base + jnp.int32(1 << b)
        cnt = jnp.sum((key >= cand).astype(jnp.int32))
        base = jnp.where(cnt >= K, cand, base)
    T = base
    gt = key > T
    need_eq = (K - jnp.sum(gt.astype(jnp.int32))).astype(jnp.float32)

    io0 = lax.broadcasted_iota(jnp.int32, (128, 128), 0)
    io1 = lax.broadcasted_iota(jnp.int32, (128, 128), 1)
    U = (io0 < io1).astype(jnp.float32)
    I128 = (io0 == io1).astype(jnp.float32)
    r0 = lax.broadcasted_iota(jnp.int32, (NROW, NROW), 0)
    r1 = lax.broadcasted_iota(jnp.int32, (NROW, NROW), 1)
    Lst = (r1 < r0).astype(jnp.float32)

    hp = jax.lax.Precision.HIGHEST

    def ex_prefix(maskf):
        inrow = jnp.dot(maskf, U, preferred_element_type=jnp.float32,
                        precision=hp)
        rowoff = jnp.dot(Lst, jnp.sum(maskf, axis=1, keepdims=True),
                         preferred_element_type=jnp.float32, precision=hp)
        return inrow + rowoff

    eq = key == T
    eq_rank = ex_prefix(eq.astype(jnp.float32))
    sel = gt | (eq & (eq_rank < need_eq))
    self_f = sel.astype(jnp.float32)
    pos = ex_prefix(self_f)
    posx_ref[...] = jnp.where(sel, pos, 3.0e6)

    acc_ref[...] = jnp.zeros((3, NSLOT), jnp.float32)
    iota_slot = lax.broadcasted_iota(
        jnp.int32, (1, NSLOT), 1).astype(jnp.float32)
    lane_io = lax.broadcasted_iota(
        jnp.int32, (1, 128), 1).astype(jnp.float32)
    hi_ref[...] = lax.shift_right_arithmetic(key, 16).astype(jnp.float32)
    lo_ref[...] = (key & jnp.int32(0xFFFF)).astype(jnp.float32)

    CH = 16

    def body(r0, carry):
        pch = posx_ref[pl.ds(r0 * CH, CH), :]
        hch = hi_ref[pl.ds(r0 * CH, CH), :]
        lch = lo_ref[pl.ds(r0 * CH, CH), :]
        pcs, hs, los, lis = [], [], [], []
        for rr in range(CH):
            prow = pch[rr:rr + 1, :]
            pcs.append(jnp.sum(I128 * prow, axis=1, keepdims=True))
            hs.append(hch[rr:rr + 1, :])
            los.append(lch[rr:rr + 1, :])
            lis.append(lane_io
                       + 128.0 * (r0 * CH + rr).astype(jnp.float32))
        pcol = jnp.concatenate(pcs, axis=0)
        oh = (pcol == iota_slot).astype(jnp.float32)
        vals = jnp.concatenate(
            [jnp.concatenate(hs, axis=1), jnp.concatenate(los, axis=1),
             jnp.concatenate(lis, axis=1)], axis=0)
        acc_ref[...] += jnp.dot(vals, oh,
                                preferred_element_type=jnp.float32,
                                precision=hp)
        return carry

    lax.fori_loop(0, NROW // CH, body, 0)

    slot_io = lax.broadcasted_iota(jnp.int32, (1, NSLOT), 1)
    valid = slot_io < K
    hic = jnp.where(valid, acc_ref[0:1, :], -32768.0)
    loc = jnp.where(valid, acc_ref[1:2, :], 0.0)
    lic = jnp.where(valid, acc_ref[2:3, :], 4.0e6 + iota_slot)

    hcols, locols, lcols = [], [], []
    for c in range(NSLOT // 128):
        sl = slice(c * 128, (c + 1) * 128)
        hcols.append(jnp.sum(I128 * hic[:, sl], axis=1, keepdims=True))
        locols.append(jnp.sum(I128 * loc[:, sl], axis=1, keepdims=True))
        lcols.append(jnp.sum(I128 * lic[:, sl], axis=1, keepdims=True))
    hcol = jnp.concatenate(hcols, axis=0)
    locol = jnp.concatenate(locols, axis=0)
    lcol = jnp.concatenate(lcols, axis=0)

    keyr = (jnp.left_shift(hic.astype(jnp.int32), 16)
            | loc.astype(jnp.int32))
    keya = (jnp.left_shift(hcol.astype(jnp.int32), 16)
            | locol.astype(jnp.int32))

    before = (keyr > keya) | ((keyr == keya) & (lic < lcol))
    ranks = jnp.sum(before.astype(jnp.float32), axis=1, keepdims=True)
    oh2 = (ranks == iota_slot).astype(jnp.float32)
    ordered = jnp.sum(oh2 * lcol, axis=0, keepdims=True)

    pid = pl.program_id(0)
    out_ref[0] = (jnp.where(valid, ordered.astype(jnp.int32), 0)
                  + pid * 20000)


def _sc_gather(nb, idx_hbm, mem_hbm, la_hbm,
               om_hbm, ola_hbm, idx_v, mem_v, la_v, sem):
    wid = lax.axis_index("s") * 2 + lax.axis_index("c")
    base = wid * nb
    pltpu.sync_copy(idx_hbm.at[pl.ds(base, nb)], idx_v)
    pltpu.async_copy(mem_hbm.at[idx_v], mem_v, sem).wait()
    pltpu.async_copy(la_hbm.at[idx_v], la_v, sem).wait()
    pltpu.sync_copy(mem_v, om_hbm.at[pl.ds(base, nb)])
    pltpu.sync_copy(la_v, ola_hbm.at[pl.ds(base, nb)])


def kernel(memory, outputs_logits, anchors, W1, b1, W2, b2, Wp, bp, topk):
    bs, N, H = memory.shape
    C = outputs_logits.shape[-1]
    f32 = jnp.float32

    scores3 = pl.pallas_call(
        _score_kernel,
        grid=(bs, N // BN),
        in_specs=[
            pl.BlockSpec((1, BN, H), lambda b, i: (b, i, 0)),
            pl.BlockSpec((1, BN, C), lambda b, i: (b, i, 0)),
            pl.BlockSpec((1, BN, 4), lambda b, i: (b, i, 0)),
            pl.BlockSpec((H, H // 2), lambda b, i: (0, 0)),
            pl.BlockSpec((1, H // 2), lambda b, i: (0, 0)),
            pl.BlockSpec((H // 2, H), lambda b, i: (0, 0)),
            pl.BlockSpec((1, H), lambda b, i: (0, 0)),
            pl.BlockSpec((H, 1), lambda b, i: (0, 0)),
            pl.BlockSpec((1, 1), lambda b, i: (0, 0)),
        ],
        out_specs=pl.BlockSpec((1, BN, 1), lambda b, i: (b, i, 0)),
        out_shape=jax.ShapeDtypeStruct((bs, N, 1), f32),
    )(memory, outputs_logits, anchors,
      W1, b1.reshape(1, -1), W2, b2.reshape(1, -1), Wp, bp.reshape(1, 1))

    spad = jnp.concatenate(
        [scores3[..., 0], jnp.full((bs, NPAD - N), -jnp.inf, f32)],
        axis=1).reshape(bs, NROW, 128)

    idx = pl.pallas_call(
        _select_kernel,
        grid=(bs,),
        in_specs=[pl.BlockSpec((1, NROW, 128), lambda b: (b, 0, 0))],
        out_specs=pl.BlockSpec((1, 1, NSLOT), lambda b: (b, 0, 0)),
        out_shape=jax.ShapeDtypeStruct((bs, 1, NSLOT), jnp.int32),
        scratch_shapes=[pltpu.VMEM((NROW, 128), f32),
                        pltpu.VMEM((NROW, 128), f32),
                        pltpu.VMEM((NROW, 128), f32),
                        pltpu.VMEM((3, NSLOT), f32)],
    )(spad)

    idx_flat = idx[:, 0, :KPAD].reshape(bs * KPAD)

    la = jnp.pad(
        jnp.concatenate([outputs_logits.reshape(bs * N, C),
                         anchors.reshape(bs * N, 4)], axis=1),
        ((0, 0), (0, 128 - C - 4)))
    nw = 32
    nb = (bs * KPAD) // nw
    mesh = plsc.VectorSubcoreMesh(core_axis_name="c", subcore_axis_name="s")
    gath = functools.partial(
        pl.kernel, mesh=mesh,
        out_type=[jax.ShapeDtypeStruct((bs * KPAD, H), f32),
                  jax.ShapeDtypeStruct((bs * KPAD, 128), f32)],
        scratch_types=[pltpu.VMEM((nb,), jnp.int32),
                       pltpu.VMEM((nb, H), f32),
                       pltpu.VMEM((nb, 128), f32),
                       pltpu.SemaphoreType.DMA],
    )(functools.partial(_sc_gather, nb))
    om, ola = gath(idx_flat, memory.reshape(bs * N, H), la)

    topk_memory = om.reshape(bs, KPAD, H)[:, :K]
    topk_logits = ola.reshape(bs, KPAD, 128)[:, :K, :C]
    topk_anchors = ola.reshape(bs, KPAD, 128)[:, :K, C:C + 4]
    return (topk_memory, topk_logits, topk_anchors)


# speedup vs baseline: 1.4025x; 1.0011x over previous
import functools

import jax
import jax.numpy as jnp
from jax import lax
from jax.experimental import pallas as pl
from jax.experimental.pallas import tpu as pltpu
from jax.experimental.pallas import tpu_sc as plsc

K = 300
KPAD = 320
NSLOT = 384
BN = 2000
NROW = 160
NPAD = NROW * 128


def _score_kernel(mem_ref, lg_ref, an_ref, w1_ref, b1_ref, w2_ref, b2_ref,
                  wp_ref, bp_ref, out_ref):
    m = mem_ref[0]
    h = jnp.maximum(
        jnp.dot(m, w1_ref[...], preferred_element_type=jnp.float32)
        + b1_ref[...], 0.0)
    proj = jnp.dot(h, w2_ref[...], preferred_element_type=jnp.float32) \
        + b2_ref[...]
    wh = jax.nn.sigmoid(an_ref[0][:, 2:4])
    area = wh[:, 0:1] * wh[:, 1:2]
    maskf = (area < 0.1).astype(jnp.float32)
    enh = m + proj * maskf
    sw = jax.nn.sigmoid(
        jnp.dot(enh, wp_ref[...], preferred_element_type=jnp.float32)
        + bp_ref[...])
    cls = jnp.max(lg_ref[0], axis=-1, keepdims=True)
    out_ref[0] = cls * (1.0 + sw * maskf)


def _select_kernel(s_ref, out_ref, posx_ref, hi_ref, lo_ref, acc_ref):
    S = jnp.maximum(s_ref[0], -3.0e38)
    bits = lax.bitcast_convert_type(S, jnp.int32)
    key = bits ^ (lax.shift_right_arithmetic(bits, 31) & jnp.int32(0x7FFFFFFF))

    base = jnp.where(jnp.sum((key >= 0).astype(jnp.int32)) >= K,
                     jnp.int32(0), jnp.int32(-2147483648))
    for b in range(30, -1, -1):
        cand = base + jnp.int32(1 << b)
        cnt = jnp.sum((key >= cand).astype(jnp.int32))
        base = jnp.where(cnt >= K, cand, base)
    T = base
    gt = key > T
    need_eq = (K - jnp.sum(gt.astype(jnp.int32))).astype(jnp.float32)

    io0 = lax.broadcasted_iota(jnp.int32, (128, 128), 0)
    io1 = lax.broadcasted_iota(jnp.int32, (128, 128), 1)
    U = (io0 < io1).astype(jnp.float32)
    I128 = (io0 == io1).astype(jnp.float32)
    r0 = lax.broadcasted_iota(jnp.int32, (NROW, NROW), 0)
    r1 = lax.broadcasted_iota(jnp.int32, (NROW, NROW), 1)
    Lst = (r1 < r0).astype(jnp.float32)

    hp = jax.lax.Precision.HIGHEST

    def ex_prefix(maskf):
        inrow = jnp.dot(maskf, U, preferred_element_type=jnp.float32,
                        precision=hp)
        rowoff = jnp.dot(Lst, jnp.sum(maskf, axis=1, keepdims=True),
                         preferred_element_type=jnp.float32, precision=hp)
        return inrow + rowoff

    eq = key == T
    eq_rank = ex_prefix(eq.astype(jnp.float32))
    sel = gt | (eq & (eq_rank < need_eq))
    self_f = sel.astype(jnp.float32)
    pos = ex_prefix(self_f)
    posx_ref[...] = jnp.where(sel, pos, 3.0e6)

    acc_ref[...] = jnp.zeros((3, NSLOT), jnp.float32)
    iota_slot = lax.broadcasted_iota(
        jnp.int32, (1, NSLOT), 1).astype(jnp.float32)
    lane_io = lax.broadcasted_iota(
        jnp.int32, (1, 128), 1).astype(jnp.float32)
    hi_ref[...] = lax.shift_right_arithmetic(key, 16).astype(jnp.float32)
    lo_ref[...] = (key & jnp.int32(0xFFFF)).astype(jnp.float32)

    CH = 16

    def body(r0, carry):
        pch = posx_ref[pl.ds(r0 * CH, CH), :]
        hch = hi_ref[pl.ds(r0 * CH, CH), :]
        lch = lo_ref[pl.ds(r0 * CH, CH), :]
        pcs, hs, los, lis = [], [], [], []
        for rr in range(CH):
            prow = pch[rr:rr + 1, :]
            pcs.append(jnp.sum(I128 * prow, axis=1, keepdims=True))
            hs.append(hch[rr:rr + 1, :])
            los.append(lch[rr:rr + 1, :])
            lis.append(lane_io
                       + 128.0 * (r0 * CH + rr).astype(jnp.float32))
        pcol = jnp.concatenate(pcs, axis=0)
        oh = (pcol == iota_slot).astype(jnp.float32)
        vals = jnp.concatenate(
            [jnp.concatenate(hs, axis=1), jnp.concatenate(los, axis=1),
             jnp.concatenate(lis, axis=1)], axis=0)
        acc_ref[...] += jnp.dot(vals, oh,
                                preferred_element_type=jnp.float32,
                                precision=hp)
        return carry

    lax.fori_loop(0, NROW // CH, body, 0)

    slot_io = lax.broadcasted_iota(jnp.int32, (1, NSLOT), 1)
    valid = slot_io < K
    hic = jnp.where(valid, acc_ref[0:1, :], -32768.0)
    loc = jnp.where(valid, acc_ref[1:2, :], 0.0)
    lic = jnp.where(valid, acc_ref[2:3, :], 4.0e6 + iota_slot)

    hcols, locols, lcols = [], [], []
    for c in range(NSLOT // 128):
        sl = slice(c * 128, (c + 1) * 128)
        hcols.append(jnp.sum(I128 * hic[:, sl], axis=1, keepdims=True))
        locols.append(jnp.sum(I128 * loc[:, sl], axis=1, keepdims=True))
        lcols.append(jnp.sum(I128 * lic[:, sl], axis=1, keepdims=True))
    hcol = jnp.concatenate(hcols, axis=0)
    locol = jnp.concatenate(locols, axis=0)
    lcol = jnp.concatenate(lcols, axis=0)

    keyr = (jnp.left_shift(hic.astype(jnp.int32), 16)
            | loc.astype(jnp.int32))
    keya = (jnp.left_shift(hcol.astype(jnp.int32), 16)
            | locol.astype(jnp.int32))

    before = (keyr > keya) | ((keyr == keya) & (lic < lcol))
    ranks = jnp.sum(before.astype(jnp.float32), axis=1, keepdims=True)
    oh2 = (ranks == iota_slot).astype(jnp.float32)
    ordered = jnp.sum(oh2 * lcol, axis=0, keepdims=True)

    pid = pl.program_id(0)
    out_ref[0] = (jnp.where(valid, ordered.astype(jnp.int32), 0)
                  + pid * 20000)


def _sc_gather(nb, idx_hbm, mem_hbm, la_hbm,
               om_hbm, ola_hbm, idx_v, mem_v, la_v, sem):
    wid = lax.axis_index("s") * 2 + lax.axis_index("c")
    base = wid * nb
    pltpu.sync_copy(idx_hbm.at[pl.ds(base, nb)], idx_v)
    pltpu.async_copy(mem_hbm.at[idx_v], mem_v, sem).wait()
    pltpu.async_copy(la_hbm.at[idx_v], la_v, sem).wait()
    pltpu.sync_copy(mem_v, om_hbm.at[pl.ds(base, nb)])
    pltpu.sync_copy(la_v, ola_hbm.at[pl.ds(base, nb)])


def kernel(memory, outputs_logits, anchors, W1, b1, W2, b2, Wp, bp, topk):
    bs, N, H = memory.shape
    C = outputs_logits.shape[-1]
    f32 = jnp.float32

    scores3 = pl.pallas_call(
        _score_kernel,
        grid=(bs, N // BN),
        in_specs=[
            pl.BlockSpec((1, BN, H), lambda b, i: (b, i, 0)),
            pl.BlockSpec((1, BN, C), lambda b, i: (b, i, 0)),
            pl.BlockSpec((1, BN, 4), lambda b, i: (b, i, 0)),
            pl.BlockSpec((H, H // 2), lambda b, i: (0, 0)),
            pl.BlockSpec((1, H // 2), lambda b, i: (0, 0)),
            pl.BlockSpec((H // 2, H), lambda b, i: (0, 0)),
            pl.BlockSpec((1, H), lambda b, i: (0, 0)),
            pl.BlockSpec((H, 1), lambda b, i: (0, 0)),
            pl.BlockSpec((1, 1), lambda b, i: (0, 0)),
        ],
        out_specs=pl.BlockSpec((1, BN, 1), lambda b, i: (b, i, 0)),
        out_shape=jax.ShapeDtypeStruct((bs, N, 1), f32),
        compiler_params=pltpu.CompilerParams(
            dimension_semantics=("parallel", "parallel")),
    )(memory, outputs_logits, anchors,
      W1, b1.reshape(1, -1), W2, b2.reshape(1, -1), Wp, bp.reshape(1, 1))

    spad = jnp.concatenate(
        [scores3[..., 0], jnp.full((bs, NPAD - N), -jnp.inf, f32)],
        axis=1).reshape(bs, NROW, 128)

    idx = pl.pallas_call(
        _select_kernel,
        grid=(bs,),
        in_specs=[pl.BlockSpec((1, NROW, 128), lambda b: (b, 0, 0))],
        out_specs=pl.BlockSpec((1, 1, NSLOT), lambda b: (b, 0, 0)),
        out_shape=jax.ShapeDtypeStruct((bs, 1, NSLOT), jnp.int32),
        scratch_shapes=[pltpu.VMEM((NROW, 128), f32),
                        pltpu.VMEM((NROW, 128), f32),
                        pltpu.VMEM((NROW, 128), f32),
                        pltpu.VMEM((3, NSLOT), f32)],
        compiler_params=pltpu.CompilerParams(
            dimension_semantics=("parallel",)),
    )(spad)

    idx_flat = idx[:, 0, :KPAD].reshape(bs * KPAD)

    la = jnp.pad(
        jnp.concatenate([outputs_logits.reshape(bs * N, C),
                         anchors.reshape(bs * N, 4)], axis=1),
        ((0, 0), (0, 128 - C - 4)))
    nw = 32
    nb = (bs * KPAD) // nw
    mesh = plsc.VectorSubcoreMesh(core_axis_name="c", subcore_axis_name="s")
    gath = functools.partial(
        pl.kernel, mesh=mesh,
        out_type=[jax.ShapeDtypeStruct((bs * KPAD, H), f32),
                  jax.ShapeDtypeStruct((bs * KPAD, 128), f32)],
        scratch_types=[pltpu.VMEM((nb,), jnp.int32),
                       pltpu.VMEM((nb, H), f32),
                       pltpu.VMEM((nb, 128), f32),
                       pltpu.SemaphoreType.DMA],
    )(functools.partial(_sc_gather, nb))
    om, ola = gath(idx_flat, memory.reshape(bs * N, H), la)

    topk_memory = om.reshape(bs, KPAD, H)[:, :K]
    topk_logits = ola.reshape(bs, KPAD, 128)[:, :K, :C]
    topk_anchors = ola.reshape(bs, KPAD, 128)[:, :K, C:C + 4]
    return (topk_memory, topk_logits, topk_anchors)
